# baseline (speedup 1.0000x reference)
import functools

import jax
import jax.numpy as jnp
from jax import lax
from jax.experimental import pallas as pl
from jax.experimental.pallas import tpu as pltpu
from jax.experimental.pallas import tpu_sc as plsc

VOCAB = 1000000
EMB = 64
BATCH = 4096
SEQ = 200
NTOK = BATCH * SEQ

NC = 2
NS = 16
NW = NC * NS
PER_W = NTOK // NW

GDMA = 128
CHUNK = 1024
IDX_ROWS = CHUNK // GDMA
N_CHUNKS = PER_W // CHUNK
IDX_ROWS_PER_W = PER_W // GDMA


@functools.partial(
    pl.kernel,
    mesh=plsc.VectorSubcoreMesh(core_axis_name="c", subcore_axis_name="s"),
    compiler_params=pltpu.CompilerParams(use_tc_tiling_on_sc=False),
    out_type=jax.ShapeDtypeStruct((NTOK, EMB), jnp.float32),
    scratch_types=[
        pltpu.VMEM((IDX_ROWS, GDMA), jnp.int32),
        pltpu.VMEM((CHUNK, EMB), jnp.float32),
        pltpu.SemaphoreType.DMA,
    ],
)
def _emb_lookup(idx_hbm, table_hbm, out_hbm, idx_v, rows_v, sem):
    wid = lax.axis_index("s") * NC + lax.axis_index("c")
    idx_row_base = wid * IDX_ROWS_PER_W
    out_base = wid * PER_W

    def body(g, carry):
        pltpu.sync_copy(idx_hbm.at[pl.ds(idx_row_base + g * IDX_ROWS, IDX_ROWS)],
                        idx_v)
        copies = [
            pltpu.async_copy(table_hbm.at[idx_v.at[j]],
                             rows_v.at[pl.ds(j * GDMA, GDMA)], sem)
            for j in range(IDX_ROWS)
        ]
        for c in copies:
            c.wait()
        pltpu.sync_copy(rows_v, out_hbm.at[pl.ds(out_base + g * CHUNK, CHUNK)])
        return carry

    lax.fori_loop(0, N_CHUNKS, body, 0)


def kernel(x, table):
    idx = x.reshape(NTOK).astype(jnp.int32).reshape(NTOK // GDMA, GDMA)
    out = _emb_lookup(idx, table)
    return out.reshape(BATCH, SEQ, EMB)


# speedup vs baseline: 1.0067x; 1.0067x over previous
import functools

import jax
import jax.numpy as jnp
from jax import lax
from jax.experimental import pallas as pl
from jax.experimental.pallas import tpu as pltpu
from jax.experimental.pallas import tpu_sc as plsc

VOCAB = 1000000
EMB = 64
BATCH = 4096
SEQ = 200
NTOK = BATCH * SEQ

NC = 2
NS = 16
NW = NC * NS
PER_W = NTOK // NW

GDMA = 128
CHUNK = 640
IDX_ROWS = CHUNK // GDMA
N_CHUNKS = PER_W // CHUNK
IDX_ROWS_PER_W = PER_W // GDMA


@functools.partial(
    pl.kernel,
    mesh=plsc.VectorSubcoreMesh(core_axis_name="c", subcore_axis_name="s"),
    compiler_params=pltpu.CompilerParams(use_tc_tiling_on_sc=False),
    out_type=jax.ShapeDtypeStruct((NTOK, EMB), jnp.float32),
    scratch_types=[
        pltpu.VMEM((2, IDX_ROWS, GDMA), jnp.int32),
        pltpu.VMEM((2, CHUNK, EMB), jnp.float32),
        pltpu.SemaphoreType.DMA,
        pltpu.SemaphoreType.DMA,
    ],
)
def _emb_lookup(idx_hbm, table_hbm, out_hbm, idx_v, rows_v, sem0, sem1):
    wid = lax.axis_index("s") * NC + lax.axis_index("c")
    idx_row_base = wid * IDX_ROWS_PER_W
    out_base = wid * PER_W
    sems = (sem0, sem1)

    def stage_and_fire(g, b):
        pltpu.sync_copy(idx_hbm.at[pl.ds(idx_row_base + g * IDX_ROWS, IDX_ROWS)],
                        idx_v.at[b])
        for j in range(IDX_ROWS):
            pltpu.async_copy(table_hbm.at[idx_v.at[b].at[j]],
                             rows_v.at[b].at[pl.ds(j * GDMA, GDMA)], sems[b])

    def drain_gathers(b):
        for j in range(IDX_ROWS):
            pltpu.make_async_copy(table_hbm.at[idx_v.at[b].at[j]],
                                  rows_v.at[b].at[pl.ds(j * GDMA, GDMA)],
                                  sems[b]).wait()

    def store_out(g, b):
        pltpu.sync_copy(rows_v.at[b],
                        out_hbm.at[pl.ds(out_base + g * CHUNK, CHUNK)])

    stage_and_fire(0, 0)
    stage_and_fire(1, 1)

    def body(p, carry):
        for b in range(2):
            g = 2 * p + b
            drain_gathers(b)
            store_out(g, b)
            stage_and_fire(g + 2, b)
        return carry

    lax.fori_loop(0, N_CHUNKS // 2 - 1, body, 0)

    for b in range(2):
        g = N_CHUNKS - 2 + b
        drain_gathers(b)
        store_out(g, b)


def kernel(x, table):
    idx = x.reshape(NTOK).astype(jnp.int32).reshape(NTOK // GDMA, GDMA)
    out = _emb_lookup(idx, table)
    return out.reshape(BATCH, SEQ, EMB)


# speedup vs baseline: 1.2239x; 1.2158x over previous
import functools

import jax
import jax.numpy as jnp
from jax import lax
from jax.experimental import pallas as pl
from jax.experimental.pallas import tpu as pltpu
from jax.experimental.pallas import tpu_sc as plsc

VOCAB = 1000000
EMB = 64
PAD_EMB = 128
BATCH = 4096
SEQ = 200
NTOK = BATCH * SEQ

NC = 2
NS = 16
NW = NC * NS
PER_W = NTOK // NW

GDMA = 128
CHUNK = 256
N_GATH = CHUNK // GDMA
N_CHUNKS = PER_W // CHUNK
LANES = 16


@functools.partial(
    pl.kernel,
    mesh=plsc.VectorSubcoreMesh(core_axis_name="c", subcore_axis_name="s"),
    out_type=jax.ShapeDtypeStruct((NTOK, EMB), jnp.float32),
    scratch_types=[
        pltpu.VMEM((2, CHUNK), jnp.int32),
        pltpu.VMEM((2, CHUNK, PAD_EMB), jnp.float32),
        pltpu.VMEM((CHUNK, EMB), jnp.float32),
        pltpu.SemaphoreType.DMA,
        pltpu.SemaphoreType.DMA,
    ],
)
def _emb_lookup(idx_hbm, table_hbm, out_hbm, idx_v, rows_v, rows64_v,
                sem0, sem1):
    wid = lax.axis_index("s") * NC + lax.axis_index("c")
    tok_base = wid * PER_W
    sems = (sem0, sem1)

    def stage_and_fire(g, b):
        pltpu.sync_copy(idx_hbm.at[pl.ds(tok_base + g * CHUNK, CHUNK)],
                        idx_v.at[b])
        for j in range(N_GATH):
            pltpu.async_copy(table_hbm.at[idx_v.at[b].at[pl.ds(j * GDMA, GDMA)]],
                             rows_v.at[b].at[pl.ds(j * GDMA, GDMA)], sems[b])

    def drain_gathers(b):
        for j in range(N_GATH):
            pltpu.make_async_copy(
                table_hbm.at[idx_v.at[b].at[pl.ds(j * GDMA, GDMA)]],
                rows_v.at[b].at[pl.ds(j * GDMA, GDMA)], sems[b]).wait()

    def compact_and_store(g, b):
        def row_body(t, carry):
            for k in range(EMB // LANES):
                rows64_v[t, pl.ds(k * LANES, LANES)] = (
                    rows_v.at[b][t, pl.ds(k * LANES, LANES)])
            return carry

        lax.fori_loop(0, CHUNK, row_body, 0)
        pltpu.sync_copy(rows64_v,
                        out_hbm.at[pl.ds(tok_base + g * CHUNK, CHUNK)])

    stage_and_fire(0, 0)
    stage_and_fire(1, 1)

    def body(p, carry):
        for b in range(2):
            g = 2 * p + b
            drain_gathers(b)
            compact_and_store(g, b)
            stage_and_fire(g + 2, b)
        return carry

    lax.fori_loop(0, N_CHUNKS // 2 - 1, body, 0)

    for b in range(2):
        g = N_CHUNKS - 2 + b
        drain_gathers(b)
        compact_and_store(g, b)


def kernel(x, table):
    idx = x.reshape(NTOK).astype(jnp.int32)
    table_padded = jnp.pad(table, ((0, 0), (0, PAD_EMB - EMB)))
    out = _emb_lookup(idx, table_padded)
    return out.reshape(BATCH, SEQ, EMB)
